# Initial kernel scaffold; baseline (speedup 1.0000x reference)
import jax
import jax.numpy as jnp
from jax.experimental import pallas as pl


def kernel(x, edge_index, W1, b1, W2, b2, W3, b3, p, Wm, bm, Wl, bl, Ww, bw):
    raise NotImplementedError("write your pallas kernel here")


# speedup vs baseline: 11.7659x; 11.7659x over previous
import functools
import jax
import jax.numpy as jnp
from jax import lax
from jax.experimental import pallas as pl
from jax.experimental.pallas import tpu as pltpu
from jax.experimental.pallas import tpu_sc as plsc

_N = 50000
_E = 800000
_NT = 50176
_EPAD = 802816
_CH = 128
_BLK = 3136
_ZR = _NT // 16

_mesh = plsc.VectorSubcoreMesh(core_axis_name="c", subcore_axis_name="s")
_SCP = pltpu.CompilerParams(use_tc_tiling_on_sc=False)


@functools.partial(
    pl.kernel,
    out_type=jax.ShapeDtypeStruct((2, _NT, 8), jnp.float32),
    mesh=_mesh,
    compiler_params=_SCP,
    scratch_types=[
        pltpu.VMEM_SHARED((_NT, 8), jnp.float32),
        pltpu.VMEM((_CH, 8), jnp.float32),
        pltpu.VMEM((_CH,), jnp.int32),
        pltpu.SemaphoreType.DMA,
    ],
)
def _sc_deg(dst_h, ones_h, zero_h, out_h, acc, ones_v, di, sem):
    c = lax.axis_index("c")
    s = lax.axis_index("s")
    per_tile = _EPAD // 32
    nch = per_tile // _CH
    pltpu.sync_copy(zero_h.at[pl.ds(s * _ZR, _ZR), :], acc.at[pl.ds(s * _ZR, _ZR), :])
    pltpu.sync_copy(ones_h, ones_v)
    plsc.subcore_barrier()
    base = (c * 16 + s) * per_tile

    def body(j, carry):
        pltpu.sync_copy(dst_h.at[pl.ds(base + j * _CH, _CH)], di)
        pltpu.sync_copy(ones_v, acc.at[di], add=True)
        return carry

    lax.fori_loop(0, nch, body, 0)
    plsc.subcore_barrier()
    pltpu.sync_copy(acc.at[pl.ds(s * _ZR, _ZR), :], out_h.at[c, pl.ds(s * _ZR, _ZR), :])


@functools.partial(
    pl.kernel,
    out_type=jax.ShapeDtypeStruct((2, _NT, 16), jnp.float32),
    mesh=_mesh,
    compiler_params=_SCP,
    scratch_types=[
        pltpu.VMEM_SHARED((_NT, 16), jnp.float32),
        pltpu.VMEM((_CH,), jnp.int32),
        pltpu.VMEM((_CH,), jnp.int32),
        pltpu.VMEM((_CH, 16), jnp.float32),
        pltpu.SemaphoreType.DMA,
    ],
)
def _sc_agg16(table_h, src_h, dst_h, zero_h, out_h, acc, si, di, rows, sem):
    c = lax.axis_index("c")
    s = lax.axis_index("s")
    per_tile = _EPAD // 32
    nch = per_tile // _CH
    pltpu.sync_copy(zero_h.at[pl.ds(s * _ZR, _ZR), :], acc.at[pl.ds(s * _ZR, _ZR), :])
    plsc.subcore_barrier()
    base = (c * 16 + s) * per_tile

    def body(j, carry):
        e0 = base + j * _CH
        pltpu.sync_copy(src_h.at[pl.ds(e0, _CH)], si)
        pltpu.sync_copy(dst_h.at[pl.ds(e0, _CH)], di)
        pltpu.async_copy(table_h.at[si], rows, sem).wait()
        pltpu.sync_copy(rows, acc.at[di], add=True)
        return carry

    lax.fori_loop(0, nch, body, 0)
    plsc.subcore_barrier()
    pltpu.sync_copy(acc.at[pl.ds(s * _ZR, _ZR), :], out_h.at[c, pl.ds(s * _ZR, _ZR), :])


@functools.partial(
    pl.kernel,
    out_type=jax.ShapeDtypeStruct((2, _NT, 32), jnp.float32),
    mesh=_mesh,
    compiler_params=_SCP,
    scratch_types=[
        pltpu.VMEM_SHARED((_NT, 32), jnp.float32),
        pltpu.VMEM((_CH,), jnp.int32),
        pltpu.VMEM((_CH,), jnp.int32),
        pltpu.VMEM((_CH, 32), jnp.float32),
        pltpu.SemaphoreType.DMA,
    ],
)
def _sc_agg32(ta_h, tb_h, src_h, dst_h, zero_h, out_h, acc, si, di, rows, sem):
    c = lax.axis_index("c")
    s = lax.axis_index("s")
    per_tile = _EPAD // 16
    nch = per_tile // _CH
    pltpu.sync_copy(zero_h.at[pl.ds(s * _ZR, _ZR), :], acc.at[pl.ds(s * _ZR, _ZR), :])
    plsc.subcore_barrier()
    base = s * per_tile

    def body(j, carry):
        e0 = base + j * _CH
        pltpu.sync_copy(src_h.at[pl.ds(e0, _CH)], si)
        pltpu.sync_copy(dst_h.at[pl.ds(e0, _CH)], di)

        @pl.when(c == 0)
        def _():
            pltpu.async_copy(ta_h.at[si], rows, sem).wait()

        @pl.when(c == 1)
        def _():
            pltpu.async_copy(tb_h.at[si], rows, sem).wait()

        pltpu.sync_copy(rows, acc.at[di], add=True)
        return carry

    lax.fori_loop(0, nch, body, 0)
    plsc.subcore_barrier()
    pltpu.sync_copy(acc.at[pl.ds(s * _ZR, _ZR), :], out_h.at[c, pl.ds(s * _ZR, _ZR), :])


def _tc_stage_a(deg_parts, xpad, W1p):
    def body(dp_ref, x_ref, w_ref, th_ref, dinv_ref):
        deg = dp_ref[0][:, 0:1] + dp_ref[1][:, 0:1] + 1.0
        dinv = lax.rsqrt(deg)
        g0 = jnp.dot(x_ref[...], w_ref[...], preferred_element_type=jnp.float32)
        th_ref[...] = g0 * dinv
        dinv_ref[...] = dinv

    return pl.pallas_call(
        body,
        grid=(16,),
        in_specs=[
            pl.BlockSpec((2, _BLK, 8), lambda i: (0, i, 0)),
            pl.BlockSpec((_BLK, 8), lambda i: (i, 0)),
            pl.BlockSpec((8, 16), lambda i: (0, 0)),
        ],
        out_specs=[
            pl.BlockSpec((_BLK, 16), lambda i: (i, 0)),
            pl.BlockSpec((_BLK, 1), lambda i: (i, 0)),
        ],
        out_shape=[
            jax.ShapeDtypeStruct((_NT, 16), jnp.float32),
            jax.ShapeDtypeStruct((_NT, 1), jnp.float32),
        ],
    )(deg_parts, xpad, W1p)


def _tc_stage_b1(u_parts, th0, dinv, b1, W2):
    def body(up_ref, t_ref, dv_ref, b_ref, w_ref, ta_ref, tb_ref):
        dv = dv_ref[...]
        h1 = jnp.maximum(dv * (up_ref[0] + up_ref[1] + t_ref[...]) + b_ref[...], 0.0)
        g1 = jnp.dot(h1, w_ref[...], preferred_element_type=jnp.float32)
        th1 = g1 * dv
        ta_ref[...] = th1[:, :32]
        tb_ref[...] = th1[:, 32:]

    return pl.pallas_call(
        body,
        grid=(16,),
        in_specs=[
            pl.BlockSpec((2, _BLK, 16), lambda i: (0, i, 0)),
            pl.BlockSpec((_BLK, 16), lambda i: (i, 0)),
            pl.BlockSpec((_BLK, 1), lambda i: (i, 0)),
            pl.BlockSpec((1, 16), lambda i: (0, 0)),
            pl.BlockSpec((16, 64), lambda i: (0, 0)),
        ],
        out_specs=[
            pl.BlockSpec((_BLK, 32), lambda i: (i, 0)),
            pl.BlockSpec((_BLK, 32), lambda i: (i, 0)),
        ],
        out_shape=[
            jax.ShapeDtypeStruct((_NT, 32), jnp.float32),
            jax.ShapeDtypeStruct((_NT, 32), jnp.float32),
        ],
    )(u_parts, th0, dinv, b1, W2)


def _tc_stage_b2(u_parts, ta, tb, dinv, b2, W3):
    def body(up_ref, ta_ref, tb_ref, dv_ref, b_ref, w_ref, oa_ref, ob_ref):
        dv = dv_ref[...]
        u = jnp.concatenate([up_ref[0], up_ref[1]], axis=1)
        t = jnp.concatenate([ta_ref[...], tb_ref[...]], axis=1)
        h2 = jnp.maximum(dv * (u + t) + b_ref[...], 0.0)
        g2 = jnp.dot(h2, w_ref[...], preferred_element_type=jnp.float32)
        th2 = g2 * dv
        oa_ref[...] = th2[:, :32]
        ob_ref[...] = th2[:, 32:]

    return pl.pallas_call(
        body,
        grid=(16,),
        in_specs=[
            pl.BlockSpec((2, _BLK, 32), lambda i: (0, i, 0)),
            pl.BlockSpec((_BLK, 32), lambda i: (i, 0)),
            pl.BlockSpec((_BLK, 32), lambda i: (i, 0)),
            pl.BlockSpec((_BLK, 1), lambda i: (i, 0)),
            pl.BlockSpec((1, 64), lambda i: (0, 0)),
            pl.BlockSpec((64, 64), lambda i: (0, 0)),
        ],
        out_specs=[
            pl.BlockSpec((_BLK, 32), lambda i: (i, 0)),
            pl.BlockSpec((_BLK, 32), lambda i: (i, 0)),
        ],
        out_shape=[
            jax.ShapeDtypeStruct((_NT, 32), jnp.float32),
            jax.ShapeDtypeStruct((_NT, 32), jnp.float32),
        ],
    )(u_parts, ta, tb, dinv, b2, W3)


def _tc_stage_b3(u_parts, ta, tb, dinv, b3, pcol):
    def body(up_ref, ta_ref, tb_ref, dv_ref, b_ref, p_ref, h3_ref, sc_ref):
        dv = dv_ref[...]
        u = jnp.concatenate([up_ref[0], up_ref[1]], axis=1)
        t = jnp.concatenate([ta_ref[...], tb_ref[...]], axis=1)
        h3 = jnp.maximum(dv * (u + t) + b_ref[...], 0.0)
        h3_ref[...] = h3
        s = jnp.dot(h3, p_ref[...], preferred_element_type=jnp.float32)
        rowid = pl.program_id(0) * _BLK + lax.broadcasted_iota(
            jnp.int32, (_BLK, 1), 0)
        sc_ref[...] = jnp.where(rowid < _N, s, -1e30)

    return pl.pallas_call(
        body,
        grid=(16,),
        in_specs=[
            pl.BlockSpec((2, _BLK, 32), lambda i: (0, i, 0)),
            pl.BlockSpec((_BLK, 32), lambda i: (i, 0)),
            pl.BlockSpec((_BLK, 32), lambda i: (i, 0)),
            pl.BlockSpec((_BLK, 1), lambda i: (i, 0)),
            pl.BlockSpec((1, 64), lambda i: (0, 0)),
            pl.BlockSpec((64, 1), lambda i: (0, 0)),
        ],
        out_specs=[
            pl.BlockSpec((_BLK, 64), lambda i: (i, 0)),
            pl.BlockSpec((_BLK, 1), lambda i: (i, 0)),
        ],
        out_shape=[
            jax.ShapeDtypeStruct((_NT, 64), jnp.float32),
            jax.ShapeDtypeStruct((_NT, 1), jnp.float32),
        ],
    )(u_parts, ta, tb, dinv, b3, pcol)


def _tc_final(score2d, h3, pcol, Wm, bm, Wl, bl):
    def body(s_ref, h3_ref, p_ref, wm_ref, bm_ref, wl_ref, bl_ref,
             mean_ref, lstd_ref, w_ref, scr):
        scr[...] = s_ref[...]
        pn = lax.rsqrt(jnp.sum(p_ref[...] * p_ref[...]))
        riota = lax.broadcasted_iota(jnp.int32, (392, 1), 0)
        ciota = lax.broadcasted_iota(jnp.int32, (1, 128), 1)
        rows = []
        for _ in range(10):
            v = scr[...]
            m = jnp.max(v)
            rowmax = jnp.max(v, axis=1, keepdims=True)
            r = jnp.min(jnp.where(rowmax >= m, riota, jnp.int32(10**9)))
            rowv = scr[pl.ds(r, 1), :]
            cc = jnp.min(jnp.where(rowv >= m, ciota, jnp.int32(10**9)))
            node = r * 128 + cc
            hrow = h3_ref[pl.ds(node, 1), :]
            rows.append(hrow * jnp.tanh(m * pn))
            scr[pl.ds(r, 1), :] = jnp.where(ciota == cc, -jnp.inf, rowv)
        hp = jnp.concatenate(rows, axis=0)
        mean_ref[...] = jnp.dot(hp, wm_ref[...],
                                preferred_element_type=jnp.float32) + bm_ref[...]
        lstd_ref[...] = jnp.dot(hp, wl_ref[...],
                                preferred_element_type=jnp.float32) + bl_ref[...]
        w_ref[...] = jnp.ones((10, 1), jnp.float32)

    return pl.pallas_call(
        body,
        out_shape=[
            jax.ShapeDtypeStruct((10, 2), jnp.float32),
            jax.ShapeDtypeStruct((10, 2), jnp.float32),
            jax.ShapeDtypeStruct((10, 1), jnp.float32),
        ],
        scratch_shapes=[pltpu.VMEM((392, 128), jnp.float32)],
    )(score2d, h3, pcol, Wm, bm, Wl, bl)


def kernel(x, edge_index, W1, b1, W2, b2, W3, b3, p, Wm, bm, Wl, bl, Ww, bw):
    f32 = jnp.float32
    src = edge_index[0].astype(jnp.int32)
    dst = edge_index[1].astype(jnp.int32)
    padn = _EPAD - _E
    srcp = jnp.concatenate([src, jnp.zeros((padn,), jnp.int32)])
    dstp = jnp.concatenate([dst, jnp.full((padn,), _N, jnp.int32)])
    xpad = jnp.zeros((_NT, 8), f32).at[:_N, :3].set(x)
    zeros8 = jnp.zeros((_NT, 8), f32)
    zeros16 = jnp.zeros((_NT, 16), f32)
    zeros32 = jnp.zeros((_NT, 32), f32)
    ones_col = jnp.ones((_CH, 8), f32)
    W1p = jnp.zeros((8, 16), f32).at[:3, :].set(W1)
    pcol = p.reshape(64, 1)

    deg_parts = _sc_deg(dstp, ones_col, zeros8)
    th0, dinv = _tc_stage_a(deg_parts, xpad, W1p)
    u0p = _sc_agg16(th0, srcp, dstp, zeros16)
    t1a, t1b = _tc_stage_b1(u0p, th0, dinv, b1.reshape(1, 16), W2)
    u1p = _sc_agg32(t1a, t1b, srcp, dstp, zeros32)
    t2a, t2b = _tc_stage_b2(u1p, t1a, t1b, dinv, b2.reshape(1, 64), W3)
    u2p = _sc_agg32(t2a, t2b, srcp, dstp, zeros32)
    h3, score = _tc_stage_b3(u2p, t2a, t2b, dinv, b3.reshape(1, 64), pcol)
    mean, lstd, w = _tc_final(score.reshape(392, 128), h3, pcol,
                              Wm, bm.reshape(1, 2), Wl, bl.reshape(1, 2))
    return mean, lstd, w


# speedup vs baseline: 23.3176x; 1.9818x over previous
import functools
import jax
import jax.numpy as jnp
from jax import lax
from jax.experimental import pallas as pl
from jax.experimental.pallas import tpu as pltpu
from jax.experimental.pallas import tpu_sc as plsc

_N = 50000
_E = 800000
_NT = 50176
_EPAD = 802816
_CH = 128
_BLK = 3136
_ZR = _NT // 16
_NCH16 = _EPAD // 32 // _CH
_NCH32 = _EPAD // 16 // _CH
_KPH = 28

_mesh = plsc.VectorSubcoreMesh(core_axis_name="c", subcore_axis_name="s")
_SCP = pltpu.CompilerParams(use_tc_tiling_on_sc=False)


@functools.partial(
    pl.kernel,
    out_type=jax.ShapeDtypeStruct((2, _NT, 8), jnp.float32),
    mesh=_mesh,
    compiler_params=_SCP,
    scratch_types=[
        pltpu.VMEM_SHARED((_NT, 8), jnp.float32),
        pltpu.VMEM((_CH, 8), jnp.float32),
        pltpu.VMEM((_CH,), jnp.int32),
        pltpu.SemaphoreType.DMA,
    ],
)
def _sc_deg(dst_h, ones_h, zero_h, out_h, acc, ones_v, di, sem):
    c = lax.axis_index("c")
    s = lax.axis_index("s")
    per_tile = _EPAD // 32
    nch = per_tile // _CH
    pltpu.sync_copy(zero_h.at[pl.ds(s * _ZR, _ZR), :], acc.at[pl.ds(s * _ZR, _ZR), :])
    pltpu.sync_copy(ones_h, ones_v)
    plsc.subcore_barrier()
    base = (c * 16 + s) * per_tile

    def body(j, carry):
        pltpu.sync_copy(dst_h.at[pl.ds(base + j * _CH, _CH)], di)
        pltpu.sync_copy(ones_v, acc.at[di], add=True)
        return carry

    lax.fori_loop(0, nch, body, 0)
    plsc.subcore_barrier()
    pltpu.sync_copy(acc.at[pl.ds(s * _ZR, _ZR), :], out_h.at[c, pl.ds(s * _ZR, _ZR), :])


@functools.partial(
    pl.kernel,
    out_type=jax.ShapeDtypeStruct((2, _NT, 16), jnp.float32),
    mesh=_mesh,
    compiler_params=_SCP,
    scratch_types=[
        pltpu.VMEM_SHARED((_NT, 16), jnp.float32),
        pltpu.VMEM((_KPH, _CH), jnp.int32),
        pltpu.VMEM((_KPH, _CH), jnp.int32),
        pltpu.VMEM((_CH, 16), jnp.float32),
        pltpu.VMEM((_CH, 16), jnp.float32),
        pltpu.SemaphoreType.DMA,
        pltpu.SemaphoreType.DMA,
    ],
)
def _sc_agg16(table_h, src2_h, dst2_h, zero_h, out_h, acc, sbuf, dbuf,
              r0, r1, g0, g1):
    rows = [r0, r1]
    gs = [g0, g1]
    c = lax.axis_index("c")
    s = lax.axis_index("s")
    crow = (c * 16 + s) * _NCH16
    pltpu.sync_copy(zero_h.at[pl.ds(s * _ZR, _ZR), :], acc.at[pl.ds(s * _ZR, _ZR), :])
    plsc.subcore_barrier()

    def phase(ph, carry):
        prow = crow + ph * _KPH
        pltpu.sync_copy(src2_h.at[pl.ds(prow, _KPH), :], sbuf)
        pltpu.sync_copy(dst2_h.at[pl.ds(prow, _KPH), :], dbuf)
        pltpu.async_copy(table_h.at[sbuf.at[0]], r0, g0)
        pltpu.async_copy(table_h.at[sbuf.at[1]], r1, g1)
        for j in range(_KPH):
            b = j % 2
            pltpu.make_async_copy(table_h.at[sbuf.at[j]], rows[b], gs[b]).wait()
            pltpu.sync_copy(rows[b], acc.at[dbuf.at[j]], add=True)
            if j + 2 < _KPH:
                pltpu.async_copy(table_h.at[sbuf.at[j + 2]], rows[b], gs[b])
        return carry

    lax.fori_loop(0, _NCH16 // _KPH, phase, 0)
    plsc.subcore_barrier()
    pltpu.sync_copy(acc.at[pl.ds(s * _ZR, _ZR), :], out_h.at[c, pl.ds(s * _ZR, _ZR), :])


@functools.partial(
    pl.kernel,
    out_type=jax.ShapeDtypeStruct((2, _NT, 32), jnp.float32),
    mesh=_mesh,
    compiler_params=_SCP,
    scratch_types=[
        pltpu.VMEM_SHARED((_NT, 32), jnp.float32),
        pltpu.VMEM((_KPH, _CH), jnp.int32),
        pltpu.VMEM((_KPH, _CH), jnp.int32),
        pltpu.VMEM((_CH, 32), jnp.float32),
        pltpu.VMEM((_CH, 32), jnp.float32),
        pltpu.SemaphoreType.DMA,
        pltpu.SemaphoreType.DMA,
    ],
)
def _sc_agg32(ta_h, tb_h, src2_h, dst2_h, zero_h, out_h, acc, sbuf, dbuf,
              r0, r1, g0, g1):
    rows = [r0, r1]
    gs = [g0, g1]
    c = lax.axis_index("c")
    s = lax.axis_index("s")
    crow = s * _NCH32
    pltpu.sync_copy(zero_h.at[pl.ds(s * _ZR, _ZR), :], acc.at[pl.ds(s * _ZR, _ZR), :])
    plsc.subcore_barrier()

    def phase(ph, carry):
        prow = crow + ph * _KPH
        pltpu.sync_copy(src2_h.at[pl.ds(prow, _KPH), :], sbuf)
        pltpu.sync_copy(dst2_h.at[pl.ds(prow, _KPH), :], dbuf)

        @pl.when(c == 0)
        def _():
            pltpu.async_copy(ta_h.at[sbuf.at[0]], r0, g0)
            pltpu.async_copy(ta_h.at[sbuf.at[1]], r1, g1)

        @pl.when(c == 1)
        def _():
            pltpu.async_copy(tb_h.at[sbuf.at[0]], r0, g0)
            pltpu.async_copy(tb_h.at[sbuf.at[1]], r1, g1)

        for j in range(_KPH):
            b = j % 2

            @pl.when(c == 0)
            def _():
                pltpu.make_async_copy(ta_h.at[sbuf.at[j]], rows[b], gs[b]).wait()

            @pl.when(c == 1)
            def _():
                pltpu.make_async_copy(tb_h.at[sbuf.at[j]], rows[b], gs[b]).wait()

            pltpu.sync_copy(rows[b], acc.at[dbuf.at[j]], add=True)
            if j + 2 < _KPH:
                @pl.when(c == 0)
                def _():
                    pltpu.async_copy(ta_h.at[sbuf.at[j + 2]], rows[b], gs[b])

                @pl.when(c == 1)
                def _():
                    pltpu.async_copy(tb_h.at[sbuf.at[j + 2]], rows[b], gs[b])
        return carry

    lax.fori_loop(0, _NCH32 // _KPH, phase, 0)
    plsc.subcore_barrier()
    pltpu.sync_copy(acc.at[pl.ds(s * _ZR, _ZR), :], out_h.at[c, pl.ds(s * _ZR, _ZR), :])


def _tc_stage_a(deg_parts, xpad, W1p):
    def body(dp_ref, x_ref, w_ref, th_ref, dinv_ref):
        deg = dp_ref[0][:, 0:1] + dp_ref[1][:, 0:1] + 1.0
        dinv = lax.rsqrt(deg)
        g0 = jnp.dot(x_ref[...], w_ref[...], preferred_element_type=jnp.float32)
        th_ref[...] = g0 * dinv
        dinv_ref[...] = dinv

    return pl.pallas_call(
        body,
        grid=(16,),
        in_specs=[
            pl.BlockSpec((2, _BLK, 8), lambda i: (0, i, 0)),
            pl.BlockSpec((_BLK, 8), lambda i: (i, 0)),
            pl.BlockSpec((8, 16), lambda i: (0, 0)),
        ],
        out_specs=[
            pl.BlockSpec((_BLK, 16), lambda i: (i, 0)),
            pl.BlockSpec((_BLK, 1), lambda i: (i, 0)),
        ],
        out_shape=[
            jax.ShapeDtypeStruct((_NT, 16), jnp.float32),
            jax.ShapeDtypeStruct((_NT, 1), jnp.float32),
        ],
    )(deg_parts, xpad, W1p)


def _tc_stage_b1(u_parts, th0, dinv, b1, W2):
    def body(up_ref, t_ref, dv_ref, b_ref, w_ref, ta_ref, tb_ref):
        dv = dv_ref[...]
        h1 = jnp.maximum(dv * (up_ref[0] + up_ref[1] + t_ref[...]) + b_ref[...], 0.0)
        g1 = jnp.dot(h1, w_ref[...], preferred_element_type=jnp.float32)
        th1 = g1 * dv
        ta_ref[...] = th1[:, :32]
        tb_ref[...] = th1[:, 32:]

    return pl.pallas_call(
        body,
        grid=(16,),
        in_specs=[
            pl.BlockSpec((2, _BLK, 16), lambda i: (0, i, 0)),
            pl.BlockSpec((_BLK, 16), lambda i: (i, 0)),
            pl.BlockSpec((_BLK, 1), lambda i: (i, 0)),
            pl.BlockSpec((1, 16), lambda i: (0, 0)),
            pl.BlockSpec((16, 64), lambda i: (0, 0)),
        ],
        out_specs=[
            pl.BlockSpec((_BLK, 32), lambda i: (i, 0)),
            pl.BlockSpec((_BLK, 32), lambda i: (i, 0)),
        ],
        out_shape=[
            jax.ShapeDtypeStruct((_NT, 32), jnp.float32),
            jax.ShapeDtypeStruct((_NT, 32), jnp.float32),
        ],
    )(u_parts, th0, dinv, b1, W2)


def _tc_stage_b2(u_parts, ta, tb, dinv, b2, W3):
    def body(up_ref, ta_ref, tb_ref, dv_ref, b_ref, w_ref, oa_ref, ob_ref):
        dv = dv_ref[...]
        u = jnp.concatenate([up_ref[0], up_ref[1]], axis=1)
        t = jnp.concatenate([ta_ref[...], tb_ref[...]], axis=1)
        h2 = jnp.maximum(dv * (u + t) + b_ref[...], 0.0)
        g2 = jnp.dot(h2, w_ref[...], preferred_element_type=jnp.float32)
        th2 = g2 * dv
        oa_ref[...] = th2[:, :32]
        ob_ref[...] = th2[:, 32:]

    return pl.pallas_call(
        body,
        grid=(16,),
        in_specs=[
            pl.BlockSpec((2, _BLK, 32), lambda i: (0, i, 0)),
            pl.BlockSpec((_BLK, 32), lambda i: (i, 0)),
            pl.BlockSpec((_BLK, 32), lambda i: (i, 0)),
            pl.BlockSpec((_BLK, 1), lambda i: (i, 0)),
            pl.BlockSpec((1, 64), lambda i: (0, 0)),
            pl.BlockSpec((64, 64), lambda i: (0, 0)),
        ],
        out_specs=[
            pl.BlockSpec((_BLK, 32), lambda i: (i, 0)),
            pl.BlockSpec((_BLK, 32), lambda i: (i, 0)),
        ],
        out_shape=[
            jax.ShapeDtypeStruct((_NT, 32), jnp.float32),
            jax.ShapeDtypeStruct((_NT, 32), jnp.float32),
        ],
    )(u_parts, ta, tb, dinv, b2, W3)


def _tc_stage_b3(u_parts, ta, tb, dinv, b3, pcol):
    def body(up_ref, ta_ref, tb_ref, dv_ref, b_ref, p_ref, h3_ref, sc_ref):
        dv = dv_ref[...]
        u = jnp.concatenate([up_ref[0], up_ref[1]], axis=1)
        t = jnp.concatenate([ta_ref[...], tb_ref[...]], axis=1)
        h3 = jnp.maximum(dv * (u + t) + b_ref[...], 0.0)
        h3_ref[...] = h3
        s = jnp.dot(h3, p_ref[...], preferred_element_type=jnp.float32)
        rowid = pl.program_id(0) * _BLK + lax.broadcasted_iota(
            jnp.int32, (_BLK, 1), 0)
        sc_ref[...] = jnp.where(rowid < _N, s, -1e30)

    return pl.pallas_call(
        body,
        grid=(16,),
        in_specs=[
            pl.BlockSpec((2, _BLK, 32), lambda i: (0, i, 0)),
            pl.BlockSpec((_BLK, 32), lambda i: (i, 0)),
            pl.BlockSpec((_BLK, 32), lambda i: (i, 0)),
            pl.BlockSpec((_BLK, 1), lambda i: (i, 0)),
            pl.BlockSpec((1, 64), lambda i: (0, 0)),
            pl.BlockSpec((64, 1), lambda i: (0, 0)),
        ],
        out_specs=[
            pl.BlockSpec((_BLK, 64), lambda i: (i, 0)),
            pl.BlockSpec((_BLK, 1), lambda i: (i, 0)),
        ],
        out_shape=[
            jax.ShapeDtypeStruct((_NT, 64), jnp.float32),
            jax.ShapeDtypeStruct((_NT, 1), jnp.float32),
        ],
    )(u_parts, ta, tb, dinv, b3, pcol)


def _tc_final(score2d, h3, pcol, Wm, bm, Wl, bl):
    def body(s_ref, h3_ref, p_ref, wm_ref, bm_ref, wl_ref, bl_ref,
             mean_ref, lstd_ref, w_ref, scr):
        scr[...] = s_ref[...]
        pn = lax.rsqrt(jnp.sum(p_ref[...] * p_ref[...]))
        riota = lax.broadcasted_iota(jnp.int32, (392, 1), 0)
        ciota = lax.broadcasted_iota(jnp.int32, (1, 128), 1)
        rows = []
        for _ in range(10):
            v = scr[...]
            m = jnp.max(v)
            rowmax = jnp.max(v, axis=1, keepdims=True)
            r = jnp.min(jnp.where(rowmax >= m, riota, jnp.int32(10**9)))
            rowv = scr[pl.ds(r, 1), :]
            cc = jnp.min(jnp.where(rowv >= m, ciota, jnp.int32(10**9)))
            node = r * 128 + cc
            hrow = h3_ref[pl.ds(node, 1), :]
            rows.append(hrow * jnp.tanh(m * pn))
            scr[pl.ds(r, 1), :] = jnp.where(ciota == cc, -jnp.inf, rowv)
        hp = jnp.concatenate(rows, axis=0)
        mean_ref[...] = jnp.dot(hp, wm_ref[...],
                                preferred_element_type=jnp.float32) + bm_ref[...]
        lstd_ref[...] = jnp.dot(hp, wl_ref[...],
                                preferred_element_type=jnp.float32) + bl_ref[...]
        w_ref[...] = jnp.ones((10, 1), jnp.float32)

    return pl.pallas_call(
        body,
        out_shape=[
            jax.ShapeDtypeStruct((10, 2), jnp.float32),
            jax.ShapeDtypeStruct((10, 2), jnp.float32),
            jax.ShapeDtypeStruct((10, 1), jnp.float32),
        ],
        scratch_shapes=[pltpu.VMEM((392, 128), jnp.float32)],
    )(score2d, h3, pcol, Wm, bm, Wl, bl)


def kernel(x, edge_index, W1, b1, W2, b2, W3, b3, p, Wm, bm, Wl, bl, Ww, bw):
    f32 = jnp.float32
    src = edge_index[0].astype(jnp.int32)
    dst = edge_index[1].astype(jnp.int32)
    padn = _EPAD - _E
    srcp = jnp.concatenate([src, jnp.zeros((padn,), jnp.int32)])
    dstp = jnp.concatenate([dst, jnp.full((padn,), _N, jnp.int32)])
    src2 = srcp.reshape(_EPAD // _CH, _CH)
    dst2 = dstp.reshape(_EPAD // _CH, _CH)
    xpad = jnp.zeros((_NT, 8), f32).at[:_N, :3].set(x)
    zeros8 = jnp.zeros((_NT, 8), f32)
    zeros16 = jnp.zeros((_NT, 16), f32)
    zeros32 = jnp.zeros((_NT, 32), f32)
    ones_col = jnp.ones((_CH, 8), f32)
    W1p = jnp.zeros((8, 16), f32).at[:3, :].set(W1)
    pcol = p.reshape(64, 1)

    deg_parts = _sc_deg(dstp, ones_col, zeros8)
    th0, dinv = _tc_stage_a(deg_parts, xpad, W1p)
    u0p = _sc_agg16(th0, src2, dst2, zeros16)
    t1a, t1b = _tc_stage_b1(u0p, th0, dinv, b1.reshape(1, 16), W2)
    u1p = _sc_agg32(t1a, t1b, src2, dst2, zeros32)
    t2a, t2b = _tc_stage_b2(u1p, t1a, t1b, dinv, b2.reshape(1, 64), W3)
    u2p = _sc_agg32(t2a, t2b, src2, dst2, zeros32)
    h3, score = _tc_stage_b3(u2p, t2a, t2b, dinv, b3.reshape(1, 64), pcol)
    mean, lstd, w = _tc_final(score.reshape(392, 128), h3, pcol,
                              Wm, bm.reshape(1, 2), Wl, bl.reshape(1, 2))
    return mean, lstd, w
